# baseline (speedup 1.0000x reference)
import jax
import jax.numpy as jnp
from jax import lax
from jax.experimental import pallas as pl
from jax.experimental.pallas import tpu as pltpu
from jax.experimental.pallas import tpu_sc as plsc

H = 200
W = 200
SH = 200.0 / 100.0
SW = 200.0 / 100.0
THRESHOLD = 0.05
TH = 8
T = H // TH
NTASKS = 2 * T * 2
P = 512
P2 = P + 16
EPS = 1e-8
CK = 128
NL = 16


def _sc_compact_kernel(raw_hbm, prm_hbm, cnt_hbm, st_v, cnt_v, raw_v, sem):
    wid = lax.axis_index("s") * 2 + lax.axis_index("c")
    lane = lax.iota(jnp.int32, NL)
    lane_f = lane.astype(jnp.float32)
    zeros_i = jnp.zeros((NL,), jnp.int32)
    zf = jnp.zeros((NL,), jnp.float32)

    for rep in range(4):
        task = wid + rep * 32

        @pl.when(task < NTASKS)
        def _run():
            bi = task // (2 * T)
            rem = task - bi * (2 * T)
            t = rem // 2
            half = rem - t * 2

            pltpu.sync_copy(raw_hbm.at[pl.ds(bi * 8, 8)], raw_v)
            for si in range(8):
                for j in range(P2 // NL):
                    st_v[si, pl.ds(j * NL, NL)] = zf

            y_lo = jnp.float32(t * TH) + 0.5
            y_hi = jnp.float32(t * TH + TH - 1) + 0.5
            x_lo = jnp.where(half == 0, jnp.float32(0.5), jnp.float32(128.5))
            x_hi = jnp.where(half == 0, jnp.float32(127.5), jnp.float32(199.5))

            def chunk(i, carry):
                off, fill, tot = carry[0], carry[1], carry[2]
                pend = carry[3:]
                sl = pl.ds(i * NL, NL)
                m_x = raw_v[0, sl]
                m_y = raw_v[1, sl]
                c0 = raw_v[2, sl]
                c1 = raw_v[3, sl]
                c3 = raw_v[4, sl]
                op = raw_v[5, sl]
                u = H / 2.0 - SH * m_y
                v = W / 2.0 - SW * m_x
                A = SH * SH * c3 + 0.3
                C = SW * SW * c0 + 0.3
                B = SH * SW * c1
                det = A * C - B * B
                det = jnp.where(det <= 1e-8, jnp.float32(1e-8), det)
                con_a = C / det
                con_b = -B / det
                con_c = A / det
                opm = jnp.where(op > THRESHOLD, op, jnp.float32(0.0))
                dmin = jnp.maximum(jnp.maximum(y_lo - u, u - y_hi),
                                   jnp.float32(0.0))
                dxm = jnp.maximum(jnp.maximum(x_lo - v, v - x_hi),
                                  jnp.float32(0.0))
                q = jnp.maximum(dmin * dmin / (A + A), dxm * dxm / (C + C))
                w = opm * jnp.exp(-q)
                mask = w > EPS
                mi = jnp.where(mask, jnp.int32(1), jnp.int32(0))
                cum = mi
                for k in (1, 2, 4, 8):
                    sh = cum.at[jnp.maximum(lane - k, 0)].get(
                        mode='promise_in_bounds')
                    cum = cum + jnp.where(lane >= k, sh, jnp.int32(0))

                def g16(x, idx):
                    return x.at[idx].get(mode='promise_in_bounds')

                lo = jnp.zeros((NL,), jnp.int32)
                for step in (8, 4, 2, 1):
                    probe = g16(cum, lo + (step - 1))
                    lo = lo + jnp.where(probe <= lane, jnp.int32(step),
                                        jnp.int32(0))
                src = jnp.minimum(lo, jnp.int32(NL - 1))
                gidx = jnp.float32(i * NL) + lane_f
                npc = cum[NL - 1]
                newc = [g16(x, src) for x in
                        (u, v, con_a, con_b, con_c, opm, gidx)]
                comb = [jnp.where(lane < fill,
                                  pend[si],
                                  g16(newc[si], jnp.maximum(lane - fill, 0)))
                        for si in range(7)]
                do_store = fill + npc >= NL

                @pl.when(do_store)
                def _flush():
                    osl = pl.ds(pl.multiple_of(off, NL), NL)
                    for si in range(7):
                        st_v[si, osl] = comb[si]

                shift = jnp.minimum(lane + (NL - fill), jnp.int32(NL - 1))
                pend2 = [jnp.where(do_store, g16(newc[si], shift), comb[si])
                         for si in range(7)]
                off2 = off + jnp.where(do_store, jnp.int32(NL), jnp.int32(0))
                fill2 = fill + npc - jnp.where(do_store, jnp.int32(NL),
                                               jnp.int32(0))
                return (off2, fill2, tot + npc) + tuple(pend2)

            def chunk_wrap(i, carry):
                return chunk(i, carry)

            init = ((jnp.int32(0), jnp.int32(0), jnp.int32(0))
                    + tuple(zf for _ in range(7)))
            fin = lax.fori_loop(0, P // NL, chunk_wrap, init)
            offf, fillf, cnt = fin[0], fin[1], fin[2]
            fsl = pl.ds(pl.multiple_of(offf, NL), NL)
            for si in range(7):
                st_v[si, fsl] = jnp.where(lane < fillf, fin[3 + si],
                                          jnp.float32(0.0))

            pltpu.sync_copy(st_v, prm_hbm.at[pl.ds(task * 8, 8)])
            cnt_v[pl.ds(0, NL)] = zeros_i + cnt
            pltpu.sync_copy(cnt_v, cnt_hbm.at[pl.ds(task * NL, NL)])


def _sc_compact(raw):
    mesh = plsc.VectorSubcoreMesh(core_axis_name="c", subcore_axis_name="s")
    f = pl.kernel(
        _sc_compact_kernel, mesh=mesh,
        out_type=[
            jax.ShapeDtypeStruct((NTASKS * 8, P2), jnp.float32),
            jax.ShapeDtypeStruct((NTASKS * NL,), jnp.int32),
        ],
        scratch_types=[
            pltpu.VMEM((8, P2), jnp.float32),
            pltpu.VMEM((NL,), jnp.int32),
            pltpu.VMEM((8, P), jnp.float32),
            pltpu.SemaphoreType.DMA,
        ],
    )
    return f(raw)


def _raster_kernel(cnt_ref, prm_ref, feat_ref, opac_ref, img_ref, cntout_ref):
    bi = pl.program_id(0)
    t = pl.program_id(1)

    op = opac_ref[0]
    maskf = (op > THRESHOLD).astype(jnp.float32)
    cntout_ref[0] = jnp.sum(maskf, axis=0, keepdims=True)

    img_ref[...] = jnp.zeros((1, img_ref.shape[1], TH, W), jnp.float32)

    feats = feat_ref[0]
    iota_p = lax.broadcasted_iota(jnp.int32, (1, P), 1).astype(jnp.float32)
    y0 = (t * TH).astype(jnp.float32) + 0.5

    for half, (c0, wh) in enumerate(((0, 128), (128, 72))):
        task = (bi * T + t) * 2 + half
        n = cnt_ref[task * NL]
        prm = prm_ref[half]
        xs = (lax.broadcasted_iota(jnp.int32, (1, wh), 1).astype(jnp.float32)
              + (c0 + 0.5))

        for c in range(P // CK):
            @pl.when(jnp.int32(c * CK) < n)
            def _chunk():
                blk = prm[c * CK:(c + 1) * CK, :]
                u = blk[:, 0:1]
                v = blk[:, 1:2]
                con_a = blk[:, 2:3]
                con_b = blk[:, 3:4]
                con_c = blk[:, 4:5]
                lopm = jnp.log(blk[:, 5:6])
                idxf = blk[:, 6:7]
                onehot = (iota_p == idxf).astype(jnp.float32)
                f = lax.dot_general(onehot, feats, (((1,), (0,)), ((), ())),
                                    preferred_element_type=jnp.float32)
                dv = xs - v
                hterm = -0.5 * (con_c * dv) * dv
                cbdv = con_b * dv
                for r in range(TH):
                    du = (y0 + float(r)) - u
                    at2 = (-0.5 * (con_a * du)) * du + lopm
                    power = (hterm + at2) - du * cbdv
                    alpha = jnp.minimum(jnp.exp(power), 0.99)
                    row = lax.dot_general(f, alpha, (((0,), (0,)), ((), ())),
                                          preferred_element_type=jnp.float32)
                    img_ref[0, :, r, c0:c0 + wh] += row


@jax.jit
def kernel(features, means3D, cov3D, opacities):
    b, p, d = features.shape
    raw = jnp.stack([
        means3D[..., 0], means3D[..., 1],
        cov3D[..., 0], cov3D[..., 1], cov3D[..., 3],
        opacities[..., 0],
        jnp.zeros((b, p), jnp.float32), jnp.zeros((b, p), jnp.float32),
    ], axis=1).reshape(b * 8, p)

    prm, cnt = _sc_compact(raw)
    prm = jnp.transpose(prm.reshape(NTASKS, 8, P2), (0, 2, 1))

    img, counts = pl.pallas_call(
        _raster_kernel,
        grid=(b, T),
        in_specs=[
            pl.BlockSpec(memory_space=pltpu.MemorySpace.SMEM),
            pl.BlockSpec((2, P2, 8), lambda bi, ti: (bi * T + ti, 0, 0)),
            pl.BlockSpec((1, p, d), lambda bi, ti: (bi, 0, 0)),
            pl.BlockSpec((1, p, 1), lambda bi, ti: (bi, 0, 0)),
        ],
        out_specs=[
            pl.BlockSpec((1, d, TH, W), lambda bi, ti: (bi, 0, ti, 0)),
            pl.BlockSpec((1, 1, 1), lambda bi, ti: (bi, 0, 0)),
        ],
        out_shape=[
            jax.ShapeDtypeStruct((b, d, H, W), jnp.float32),
            jax.ShapeDtypeStruct((b, 1, 1), jnp.float32),
        ],
    )(cnt, prm, features, opacities)
    return img, jnp.mean(counts)


# speedup vs baseline: 1.4965x; 1.4965x over previous
import jax
import jax.numpy as jnp
from jax import lax
from jax.experimental import pallas as pl
from jax.experimental.pallas import tpu as pltpu
from jax.experimental.pallas import tpu_sc as plsc

H = 200
W = 200
SH = 200.0 / 100.0
SW = 200.0 / 100.0
THRESHOLD = 0.05
TH = 8
T = H // TH
NTASKS = 2 * T
P = 512
P2 = P + 16
EPS = 1e-8
CK = 128
NL = 16


def _sc_compact_kernel(raw_hbm, prm_hbm, cnt_hbm, st_v, cnt_v, raw_v, sem):
    wid = lax.axis_index("s") * 2 + lax.axis_index("c")
    lane = lax.iota(jnp.int32, NL)
    lane_f = lane.astype(jnp.float32)
    zeros_i = jnp.zeros((NL,), jnp.int32)
    zf = jnp.zeros((NL,), jnp.float32)

    for rep in range(2):
        task = wid + rep * 32

        @pl.when(task < NTASKS)
        def _run():
            bi = task // T
            t = task - bi * T

            pltpu.sync_copy(raw_hbm.at[pl.ds(bi * 8, 8)], raw_v)
            for si in range(8):
                for j in range(P2 // NL):
                    st_v[si, pl.ds(j * NL, NL)] = zf

            y_lo = jnp.float32(t * TH) + 0.5
            y_hi = jnp.float32(t * TH + TH - 1) + 0.5

            def chunk(i, carry):
                off, fill, tot = carry[0], carry[1], carry[2]
                pend = carry[3:]
                sl = pl.ds(i * NL, NL)
                m_x = raw_v[0, sl]
                m_y = raw_v[1, sl]
                c0 = raw_v[2, sl]
                c1 = raw_v[3, sl]
                c3 = raw_v[4, sl]
                op = raw_v[5, sl]
                u = H / 2.0 - SH * m_y
                v = W / 2.0 - SW * m_x
                A = SH * SH * c3 + 0.3
                C = SW * SW * c0 + 0.3
                B = SH * SW * c1
                det = A * C - B * B
                det = jnp.where(det <= 1e-8, jnp.float32(1e-8), det)
                con_a = C / det
                con_b = -B / det
                con_c = A / det
                opm = jnp.where(op > THRESHOLD, op, jnp.float32(0.0))
                dmin = jnp.maximum(jnp.maximum(y_lo - u, u - y_hi),
                                   jnp.float32(0.0))
                q = dmin * dmin / (A + A)
                w = opm * jnp.exp(-q)
                mask = w > EPS
                mi = jnp.where(mask, jnp.int32(1), jnp.int32(0))
                cum = mi
                for k in (1, 2, 4, 8):
                    sh = cum.at[jnp.maximum(lane - k, 0)].get(
                        mode='promise_in_bounds')
                    cum = cum + jnp.where(lane >= k, sh, jnp.int32(0))

                def g16(x, idx):
                    return x.at[idx].get(mode='promise_in_bounds')

                lo = jnp.zeros((NL,), jnp.int32)
                for step in (8, 4, 2, 1):
                    probe = g16(cum, lo + (step - 1))
                    lo = lo + jnp.where(probe <= lane, jnp.int32(step),
                                        jnp.int32(0))
                src = jnp.minimum(lo, jnp.int32(NL - 1))
                gidx = jnp.float32(i * NL) + lane_f
                npc = cum[NL - 1]
                newc = [g16(x, src) for x in
                        (u, v, con_a, con_b, con_c, opm, gidx)]
                comb = [jnp.where(lane < fill,
                                  pend[si],
                                  g16(newc[si], jnp.maximum(lane - fill, 0)))
                        for si in range(7)]
                do_store = fill + npc >= NL

                @pl.when(do_store)
                def _flush():
                    osl = pl.ds(pl.multiple_of(off, NL), NL)
                    for si in range(7):
                        st_v[si, osl] = comb[si]

                shift = jnp.minimum(lane + (NL - fill), jnp.int32(NL - 1))
                pend2 = [jnp.where(do_store, g16(newc[si], shift), comb[si])
                         for si in range(7)]
                off2 = off + jnp.where(do_store, jnp.int32(NL), jnp.int32(0))
                fill2 = fill + npc - jnp.where(do_store, jnp.int32(NL),
                                               jnp.int32(0))
                return (off2, fill2, tot + npc) + tuple(pend2)

            def chunk_wrap(i, carry):
                return chunk(i, carry)

            init = ((jnp.int32(0), jnp.int32(0), jnp.int32(0))
                    + tuple(zf for _ in range(7)))
            fin = lax.fori_loop(0, P // NL, chunk_wrap, init)
            offf, fillf, cnt = fin[0], fin[1], fin[2]
            fsl = pl.ds(pl.multiple_of(offf, NL), NL)
            for si in range(7):
                st_v[si, fsl] = jnp.where(lane < fillf, fin[3 + si],
                                          jnp.float32(0.0))

            pltpu.sync_copy(st_v, prm_hbm.at[pl.ds(task * 8, 8)])
            cnt_v[pl.ds(0, NL)] = zeros_i + cnt
            pltpu.sync_copy(cnt_v, cnt_hbm.at[pl.ds(task * NL, NL)])


def _sc_compact(raw):
    mesh = plsc.VectorSubcoreMesh(core_axis_name="c", subcore_axis_name="s")
    f = pl.kernel(
        _sc_compact_kernel, mesh=mesh,
        out_type=[
            jax.ShapeDtypeStruct((NTASKS * 8, P2), jnp.float32),
            jax.ShapeDtypeStruct((NTASKS * NL,), jnp.int32),
        ],
        scratch_types=[
            pltpu.VMEM((8, P2), jnp.float32),
            pltpu.VMEM((NL,), jnp.int32),
            pltpu.VMEM((8, P), jnp.float32),
            pltpu.SemaphoreType.DMA,
        ],
    )
    return f(raw)


def _raster_kernel(cnt_ref, prm_ref, feat_ref, opac_ref, img_ref, cntout_ref):
    bi = pl.program_id(0)
    t = pl.program_id(1)
    task = bi * T + t

    op = opac_ref[0]
    maskf = (op > THRESHOLD).astype(jnp.float32)
    cntout_ref[0] = jnp.sum(maskf, axis=0, keepdims=True)

    img_ref[...] = jnp.zeros((1, img_ref.shape[1], TH, W), jnp.float32)

    n = cnt_ref[task * NL]
    prm = prm_ref[0]
    feats = feat_ref[0]
    xs = lax.broadcasted_iota(jnp.int32, (1, W), 1).astype(jnp.float32) + 0.5
    iota_p = lax.broadcasted_iota(jnp.int32, (1, P), 1).astype(jnp.float32)
    y0 = (t * TH).astype(jnp.float32) + 0.5

    for c in range(P // CK):
        @pl.when(jnp.int32(c * CK) < n)
        def _chunk():
            blk = prm[c * CK:(c + 1) * CK, :]
            u = blk[:, 0:1]
            v = blk[:, 1:2]
            con_a = blk[:, 2:3]
            con_b = blk[:, 3:4]
            con_c = blk[:, 4:5]
            lopm = jnp.log(blk[:, 5:6])
            idxf = blk[:, 6:7]
            onehot = (iota_p == idxf).astype(jnp.float32)
            f = lax.dot_general(onehot, feats, (((1,), (0,)), ((), ())),
                                preferred_element_type=jnp.float32)
            dv = xs - v
            hterm = -0.5 * (con_c * dv) * dv
            cbdv = con_b * dv
            for r in range(TH):
                du = (y0 + float(r)) - u
                at2 = (-0.5 * (con_a * du)) * du + lopm
                power = (hterm + at2) - du * cbdv
                alpha = jnp.minimum(jnp.exp(power), 0.99)
                row = lax.dot_general(f, alpha, (((0,), (0,)), ((), ())),
                                      preferred_element_type=jnp.float32)
                img_ref[0, :, r, :] += row


@jax.jit
def kernel(features, means3D, cov3D, opacities):
    b, p, d = features.shape
    raw = jnp.stack([
        means3D[..., 0], means3D[..., 1],
        cov3D[..., 0], cov3D[..., 1], cov3D[..., 3],
        opacities[..., 0],
        jnp.zeros((b, p), jnp.float32), jnp.zeros((b, p), jnp.float32),
    ], axis=1).reshape(b * 8, p)

    prm, cnt = _sc_compact(raw)
    prm = jnp.transpose(prm.reshape(NTASKS, 8, P2), (0, 2, 1))

    img, counts = pl.pallas_call(
        _raster_kernel,
        grid=(b, T),
        in_specs=[
            pl.BlockSpec(memory_space=pltpu.MemorySpace.SMEM),
            pl.BlockSpec((1, P2, 8), lambda bi, ti: (bi * T + ti, 0, 0)),
            pl.BlockSpec((1, p, d), lambda bi, ti: (bi, 0, 0)),
            pl.BlockSpec((1, p, 1), lambda bi, ti: (bi, 0, 0)),
        ],
        out_specs=[
            pl.BlockSpec((1, d, TH, W), lambda bi, ti: (bi, 0, ti, 0)),
            pl.BlockSpec((1, 1, 1), lambda bi, ti: (bi, 0, 0)),
        ],
        out_shape=[
            jax.ShapeDtypeStruct((b, d, H, W), jnp.float32),
            jax.ShapeDtypeStruct((b, 1, 1), jnp.float32),
        ],
    )(cnt, prm, features, opacities)
    return img, jnp.mean(counts)
